# baseline (speedup 1.0000x reference)
import functools
import math

import jax
import jax.numpy as jnp
from jax import lax
from jax.experimental import pallas as pl
from jax.experimental.pallas import tpu as pltpu
from jax.experimental.pallas import tpu_sc as plsc

EMB = 64
SCALE = math.sqrt(EMB)
NBUF = 4
LEAD = 2


def kernel(tokens, table):
    B, L = tokens.shape
    vocab, emb = table.shape
    assert emb == EMB
    info = plsc.get_sparse_core_info()
    num_workers = info.num_cores * info.num_subcores
    assert B % num_workers == 0
    rows_per_w = B // num_workers
    assert rows_per_w % NBUF == 0 and rows_per_w >= 2 * NBUF

    tokens = tokens.astype(jnp.int32)

    mesh = plsc.VectorSubcoreMesh(core_axis_name="c", subcore_axis_name="s")

    @functools.partial(
        pl.kernel,
        out_type=jax.ShapeDtypeStruct((B, L, EMB), jnp.float32),
        mesh=mesh,
        scratch_types=[
            pltpu.VMEM((rows_per_w, L), jnp.int32),
            [pltpu.VMEM((L, EMB), jnp.float32) for _ in range(NBUF)],
            [pltpu.SemaphoreType.DMA for _ in range(NBUF)],
            [pltpu.SemaphoreType.DMA for _ in range(NBUF)],
        ],
        compiler_params=pltpu.CompilerParams(use_tc_tiling_on_sc=False),
    )
    def emb_lookup(tok_hbm, table_hbm, out_hbm, idx_all, rows, gsem, wsem):
        wid = lax.axis_index("s") * info.num_cores + lax.axis_index("c")
        base = wid * rows_per_w

        pltpu.sync_copy(tok_hbm.at[pl.ds(base, rows_per_w)], idx_all)

        def fire_gather(r, b):
            pltpu.async_copy(table_hbm.at[idx_all.at[r]], rows[b], gsem[b])

        def wait_gather(r, b):
            pltpu.make_async_copy(
                table_hbm.at[idx_all.at[r]], rows[b], gsem[b]
            ).wait()

        def fire_wb(r, b):
            pltpu.async_copy(rows[b], out_hbm.at[base + r], wsem[b])

        def wait_wb(r, b):
            pltpu.make_async_copy(rows[b], out_hbm.at[base + r], wsem[b]).wait()

        def scale(rref):
            def row(i, carry):
                for k in range(EMB // 16):
                    sl = pl.ds(k * 16, 16)
                    rref[i, sl] = rref[i, sl] * SCALE
                return carry

            lax.fori_loop(0, L, row, 0, unroll=5)

        def process(r, b):
            wait_gather(r, b)
            scale(rows[b])
            fire_wb(r, b)

        for r in range(LEAD):
            fire_gather(r, r % NBUF)
        for r in range(LEAD):
            process(r, r % NBUF)
            fire_gather(r + LEAD, (r + LEAD) % NBUF)

        def outer(i, carry):
            r0 = LEAD + i * NBUF
            for j in range(NBUF):
                r = r0 + j
                b = (LEAD + j) % NBUF
                process(r, b)
                br = j % NBUF
                wait_wb(r - LEAD, br)
                fire_gather(r + LEAD, br)
            return carry

        lax.fori_loop(0, (rows_per_w - 2 * LEAD) // NBUF, outer, 0)

        for r in range(rows_per_w - LEAD, rows_per_w):
            process(r, r % NBUF)
        for r in range(rows_per_w - NBUF, rows_per_w):
            wait_wb(r, r % NBUF)

    return emb_lookup(tokens, table)
